# Initial kernel scaffold; baseline (speedup 1.0000x reference)
import functools

import jax
import jax.numpy as jnp
from jax import lax
from jax.experimental import pallas as pl
from jax.experimental.pallas import tpu as pltpu
from jax.experimental.pallas import tpu_sc as plsc

VOCAB = 1000000
EMB = 16
BATCH = 16384
SEQ = 200
SEQP = 208
LANES = 128
NW = 32
ROWS_W = BATCH // NW
CHUNK_ROWS = 16
N_CHUNKS = ROWS_W // CHUNK_ROWS
IDXR = CHUNK_ROWS * SEQP // LANES
IDXR_W = ROWS_W * SEQP // LANES
TOT_IDXR = BATCH * SEQP // LANES


def _dtable_body(emb_ref, m_ref, out_ref):
    out_ref[...] = jnp.dot(emb_ref[...], m_ref[...],
                           preferred_element_type=jnp.float32,
                           precision=lax.Precision.HIGHEST)


def _make_dtable(emb128, m):
    blk = 5000
    return pl.pallas_call(
        _dtable_body,
        grid=(emb128.shape[0] // blk,),
        in_specs=[pl.BlockSpec((blk, 128), lambda i: (i, 0)),
                  pl.BlockSpec((128, 8), lambda i: (0, 0))],
        out_specs=pl.BlockSpec((blk, 8), lambda i: (i, 0)),
        out_shape=jax.ShapeDtypeStruct((emb128.shape[0], 8), jnp.float32),
    )(emb128, m)


def _sc_body(ids_hbm, dt_hbm, consts_hbm, out_hbm,
             idx_v, g_v, stage_v, out_v, consts_v, d0_v, sem):
    c = lax.axis_index("c")
    s = lax.axis_index("s")
    wid = s * 2 + c
    idx_base = wid * IDXR_W

    pltpu.sync_copy(consts_hbm, consts_v)
    pltpu.sync_copy(dt_hbm.at[pl.ds(0, 16)], d0_v)
    lane0 = (lax.iota(jnp.int32, 16) == 0).astype(jnp.float32)
    bdiff = lax.reduce_sum_p.bind(consts_v[...] * lane0, axes=(0,))
    d0 = lax.reduce_sum_p.bind(d0_v[...] * lane0, axes=(0,))
    corr = bdiff - 8.0 * d0

    row_ids = lax.iota(jnp.int32, 16)
    col0 = row_ids * 0
    col1 = col0 + 1

    def chunk_step(k, carry):
        pltpu.sync_copy(ids_hbm.at[pl.ds(idx_base + k * IDXR, IDXR)], idx_v)
        pltpu.async_copy(dt_hbm.at[idx_v], g_v, sem).wait()

        for r in range(CHUNK_ROWS):
            acc = None
            for j in range(13):
                flat = r * SEQP + j * 16
                v = g_v[flat // LANES, pl.ds(flat % LANES, 16)]
                acc = v if acc is None else acc + v
            stage_v[pl.ds(r * 16, 16)] = acc

        rowsum = None
        base16 = lax.iota(jnp.int32, 16) * 16
        for col in range(16):
            part = plsc.load_gather(stage_v, [base16 + col])
            rowsum = part if rowsum is None else rowsum + part
        z = rowsum + corr
        p0 = 1.0 / (1.0 + jnp.exp(-z))
        p1 = 1.0 - p0
        rows = k * CHUNK_ROWS + row_ids
        plsc.store_scatter(out_v, [rows, col0], p0)
        plsc.store_scatter(out_v, [rows, col1], p1)
        return carry

    lax.fori_loop(0, N_CHUNKS, chunk_step, 0)
    pltpu.sync_copy(out_v, out_hbm.at[pl.ds(wid * ROWS_W, ROWS_W)])


@functools.partial(
    pl.kernel,
    mesh=plsc.VectorSubcoreMesh(core_axis_name="c", subcore_axis_name="s"),
    out_type=jax.ShapeDtypeStruct((BATCH, 2), jnp.float32),
    scratch_types=[
        pltpu.VMEM((IDXR, LANES), jnp.int32),
        pltpu.VMEM((IDXR, LANES), jnp.float32),
        pltpu.VMEM((256,), jnp.float32),
        pltpu.VMEM((ROWS_W, 2), jnp.float32),
        pltpu.VMEM((16,), jnp.float32),
        pltpu.VMEM((16,), jnp.float32),
        pltpu.SemaphoreType.DMA,
    ],
)
def _sc_kernel(ids_hbm, dt_hbm, consts_hbm, out_hbm, *scratch):
    _sc_body(ids_hbm, dt_hbm, consts_hbm, out_hbm, *scratch)


def kernel(input_ids, emb_table, W, b):
    wdiff = (W[0] - W[1]) * (1.0 / SEQ)
    m = jnp.kron(jnp.eye(8, dtype=jnp.float32), wdiff[:, None])
    emb128 = emb_table.reshape(VOCAB // 8, 128)
    dtable = _make_dtable(emb128, m).reshape(VOCAB)

    ids = input_ids.astype(jnp.int32)
    ids_pad = jnp.concatenate(
        [ids, jnp.zeros((BATCH, SEQP - SEQ), jnp.int32)], axis=1)
    ids128 = ids_pad.reshape(TOT_IDXR, LANES)

    consts = jnp.zeros((16,), jnp.float32).at[0].set(b[0] - b[1])
    return _sc_kernel(ids128, dtable, consts)


# speedup vs baseline: 7.0915x; 7.0915x over previous
import functools

import jax
import jax.numpy as jnp
from jax import lax
from jax.experimental import pallas as pl
from jax.experimental.pallas import tpu as pltpu
from jax.experimental.pallas import tpu_sc as plsc

VOCAB = 1000000
EMB = 16
BATCH = 16384
SEQ = 200
NW = 32
ROWS_W = BATCH // NW
CHUNK_ROWS = 16
N_CHUNKS = ROWS_W // CHUNK_ROWS
CHUNK_IDS = CHUNK_ROWS * SEQ
IDS_W = ROWS_W * SEQ


def _dtable_body(emb_ref, m_ref, out_ref):
    out_ref[...] = jnp.dot(emb_ref[...], m_ref[...],
                           preferred_element_type=jnp.float32,
                           precision=lax.Precision.HIGHEST)


def _make_dtable(emb128, m):
    blk = 5000
    return pl.pallas_call(
        _dtable_body,
        grid=(emb128.shape[0] // blk,),
        in_specs=[pl.BlockSpec((blk, 128), lambda i: (i, 0)),
                  pl.BlockSpec((128, 8), lambda i: (0, 0))],
        out_specs=pl.BlockSpec((blk, 8), lambda i: (i, 0)),
        out_shape=jax.ShapeDtypeStruct((emb128.shape[0], 8), jnp.float32),
    )(emb128, m)


def _sc_body(ids_hbm, dt_hbm, consts_hbm, out0_hbm, out1_hbm,
             idx_v, g_v, out0_v, out1_v, consts_v, sem):
    c = lax.axis_index("c")
    s = lax.axis_index("s")
    wid = s * 2 + c
    idx_base = wid * IDS_W

    pltpu.sync_copy(consts_hbm, consts_v)
    corr = consts_v[...]

    def chunk_step(k, carry):
        pltpu.sync_copy(
            ids_hbm.at[pl.ds(idx_base + k * CHUNK_IDS, CHUNK_IDS)], idx_v)
        pltpu.async_copy(dt_hbm.at[idx_v], g_v, sem).wait()

        acc = g_v[pl.ds(0, 16)]
        for t in range(1, SEQ):
            acc = acc + g_v[pl.ds(t * 16, 16)]

        z = acc + corr
        p0 = 1.0 / (1.0 + jnp.exp(-z))
        p1 = 1.0 - p0
        out0_v[pl.ds(k * CHUNK_ROWS, CHUNK_ROWS)] = p0
        out1_v[pl.ds(k * CHUNK_ROWS, CHUNK_ROWS)] = p1
        return carry

    lax.fori_loop(0, N_CHUNKS, chunk_step, 0)
    pltpu.sync_copy(out0_v, out0_hbm.at[pl.ds(wid * ROWS_W, ROWS_W)])
    pltpu.sync_copy(out1_v, out1_hbm.at[pl.ds(wid * ROWS_W, ROWS_W)])


@functools.partial(
    pl.kernel,
    mesh=plsc.VectorSubcoreMesh(core_axis_name="c", subcore_axis_name="s"),
    out_type=(jax.ShapeDtypeStruct((BATCH,), jnp.float32),
              jax.ShapeDtypeStruct((BATCH,), jnp.float32)),
    scratch_types=[
        pltpu.VMEM((CHUNK_IDS,), jnp.int32),
        pltpu.VMEM((CHUNK_IDS,), jnp.float32),
        pltpu.VMEM((ROWS_W,), jnp.float32),
        pltpu.VMEM((ROWS_W,), jnp.float32),
        pltpu.VMEM((16,), jnp.float32),
        pltpu.SemaphoreType.DMA,
    ],
)
def _sc_kernel(ids_hbm, dt_hbm, consts_hbm, out0_hbm, out1_hbm, *scratch):
    _sc_body(ids_hbm, dt_hbm, consts_hbm, out0_hbm, out1_hbm, *scratch)


def kernel(input_ids, emb_table, W, b):
    wdiff = (W[0] - W[1]) * (1.0 / SEQ)
    m = jnp.kron(jnp.eye(8, dtype=jnp.float32), wdiff[:, None])
    emb128 = emb_table.reshape(VOCAB // 8, 128)
    dtable = _make_dtable(emb128, m).reshape(VOCAB)

    ids = input_ids.astype(jnp.int32)
    ids_t = ids.reshape(BATCH // 16, 16, SEQ).transpose(0, 2, 1)
    ids_flat = ids_t.reshape(BATCH * SEQ)

    consts = jnp.full((16,), b[0] - b[1], jnp.float32)
    p0, p1 = _sc_kernel(ids_flat, dtable, consts)
    return jnp.stack([p0, p1], axis=1)


# speedup vs baseline: 7.7341x; 1.0906x over previous
import functools

import jax
import jax.numpy as jnp
from jax import lax
from jax.experimental import pallas as pl
from jax.experimental.pallas import tpu as pltpu
from jax.experimental.pallas import tpu_sc as plsc

VOCAB = 1000000
EMB = 16
BATCH = 16384
SEQ = 200
NW = 32
ROWS_W = BATCH // NW
CHUNK_ROWS = 128
N_CHUNKS = ROWS_W // CHUNK_ROWS
CHUNK_IDS = CHUNK_ROWS * SEQ
IDS_W = ROWS_W * SEQ


def _dtable_body(emb_ref, m_ref, out_ref):
    out_ref[...] = jnp.dot(emb_ref[...], m_ref[...],
                           preferred_element_type=jnp.float32,
                           precision=lax.Precision.HIGHEST)


def _make_dtable(emb128, m):
    blk = 5000
    return pl.pallas_call(
        _dtable_body,
        grid=(emb128.shape[0] // blk,),
        in_specs=[pl.BlockSpec((blk, 128), lambda i: (i, 0)),
                  pl.BlockSpec((128, 8), lambda i: (0, 0))],
        out_specs=pl.BlockSpec((blk, 8), lambda i: (i, 0)),
        out_shape=jax.ShapeDtypeStruct((emb128.shape[0], 8), jnp.float32),
    )(emb128, m)


def _tr_body(ids_ref, out_ref):
    x = ids_ref[...]
    xf = lax.bitcast_convert_type(x, jnp.float32)
    out_ref[...] = lax.bitcast_convert_type(xf.T, jnp.int32)[None]


def _transpose_ids(ids):
    g = BATCH // CHUNK_ROWS
    out = pl.pallas_call(
        _tr_body,
        grid=(g,),
        in_specs=[pl.BlockSpec((CHUNK_ROWS, SEQ), lambda i: (i, 0))],
        out_specs=pl.BlockSpec((1, SEQ, CHUNK_ROWS), lambda i: (i, 0, 0)),
        out_shape=jax.ShapeDtypeStruct((g, SEQ, CHUNK_ROWS), jnp.int32),
    )(ids)
    return out.reshape(BATCH * SEQ)


def _sc_body(ids_hbm, dt_hbm, consts_hbm, out0_hbm, out1_hbm,
             idx_v, g_v, out0_v, out1_v, consts_v, sem):
    c = lax.axis_index("c")
    s = lax.axis_index("s")
    wid = s * 2 + c
    idx_base = wid * IDS_W

    pltpu.sync_copy(consts_hbm, consts_v)
    corr = consts_v[...]

    def chunk_step(k, carry):
        pltpu.sync_copy(
            ids_hbm.at[pl.ds(idx_base + k * CHUNK_IDS, CHUNK_IDS)], idx_v)
        pltpu.async_copy(dt_hbm.at[idx_v], g_v, sem).wait()

        for j in range(8):
            def tok_step(t, acc, j=j):
                return acc + g_v[pl.ds(t * CHUNK_ROWS + j * 16, 16)]
            z = lax.fori_loop(1, SEQ, tok_step, g_v[pl.ds(j * 16, 16)])
            z = z + corr
            p0 = 1.0 / (1.0 + jnp.exp(-z))
            out0_v[pl.ds(k * CHUNK_ROWS + j * 16, 16)] = p0
            out1_v[pl.ds(k * CHUNK_ROWS + j * 16, 16)] = 1.0 - p0
        return carry

    lax.fori_loop(0, N_CHUNKS, chunk_step, 0)
    pltpu.sync_copy(out0_v, out0_hbm.at[pl.ds(wid * ROWS_W, ROWS_W)])
    pltpu.sync_copy(out1_v, out1_hbm.at[pl.ds(wid * ROWS_W, ROWS_W)])


@functools.partial(
    pl.kernel,
    mesh=plsc.VectorSubcoreMesh(core_axis_name="c", subcore_axis_name="s"),
    out_type=(jax.ShapeDtypeStruct((BATCH,), jnp.float32),
              jax.ShapeDtypeStruct((BATCH,), jnp.float32)),
    scratch_types=[
        pltpu.VMEM((CHUNK_IDS,), jnp.int32),
        pltpu.VMEM((CHUNK_IDS,), jnp.float32),
        pltpu.VMEM((ROWS_W,), jnp.float32),
        pltpu.VMEM((ROWS_W,), jnp.float32),
        pltpu.VMEM((16,), jnp.float32),
        pltpu.SemaphoreType.DMA,
    ],
)
def _sc_kernel(ids_hbm, dt_hbm, consts_hbm, out0_hbm, out1_hbm, *scratch):
    _sc_body(ids_hbm, dt_hbm, consts_hbm, out0_hbm, out1_hbm, *scratch)


def kernel(input_ids, emb_table, W, b):
    wdiff = (W[0] - W[1]) * (1.0 / SEQ)
    m = jnp.kron(jnp.eye(8, dtype=jnp.float32), wdiff[:, None])
    emb128 = emb_table.reshape(VOCAB // 8, 128)
    dtable = _make_dtable(emb128, m).reshape(VOCAB)

    ids_flat = _transpose_ids(input_ids.astype(jnp.int32))

    consts = jnp.full((16,), b[0] - b[1], jnp.float32)
    p0, p1 = _sc_kernel(ids_flat, dtable, consts)
    return jnp.stack([p0, p1], axis=1)
